# baseline (speedup 1.0000x reference)
import jax
import jax.numpy as jnp
from jax import lax
from jax.experimental import pallas as pl
from jax.experimental.pallas import tpu as pltpu
from jax.experimental.pallas import tpu_sc as plsc

N_NUM = 13
N_CAT = 26
CARD = 1000
D = 128
B = 4096
N_TOK = N_NUM + N_CAT

try:
    _INFO = plsc.get_sparse_core_info()
    _NC = _INFO.num_cores
    _NS = _INFO.num_subcores
except Exception:
    _NC = 2
    _NS = 16
_NW = _NC * _NS
_CH = 256
_CPT = (N_CAT * B) // _CH // _NW
_NBUF = 3


def _fold_body(tab_ref, bias_ref, out_ref):
    out_ref[:CARD] = tab_ref[:CARD] + bias_ref[0]
    out_ref[CARD:] = tab_ref[CARD:] + bias_ref[1]


def _fold_table(cat_table, cat_bias):
    return pl.pallas_call(
        _fold_body,
        grid=(N_CAT // 2,),
        in_specs=[
            pl.BlockSpec((2 * CARD, D), lambda i: (i, 0)),
            pl.BlockSpec((2, 1, D), lambda i: (i, 0, 0)),
        ],
        out_specs=pl.BlockSpec((2 * CARD, D), lambda i: (i, 0)),
        out_shape=jax.ShapeDtypeStruct((N_CAT * CARD, D), jnp.float32),
    )(cat_table, cat_bias[:, None, :])


def _sc_body(xcatt_hbm, tab_hbm, out_hbm,
             ix0, ix1, ix2, sb0, sb1, sb2,
             sg0, sg1, sg2, so0, so1, so2):
    wid = lax.axis_index("s") * _NC + lax.axis_index("c")
    k0 = wid * _CPT

    ixs = [ix0, ix1, ix2]
    sbs = [sb0, sb1, sb2]
    sgs = [sg0, sg1, sg2]
    sos = [so0, so1, so2]

    def chunk_coords(j):
        k = k0 + j
        f = lax.div(k, jnp.int32(B // _CH))
        b0 = lax.rem(k, jnp.int32(B // _CH)) * _CH
        return f, b0

    def start_gather(j, b):
        f, b0 = chunk_coords(j)
        pltpu.sync_copy(xcatt_hbm.at[f, pl.ds(b0, _CH)], ixs[b])
        for v in range(_CH // 16):
            s = pl.ds(v * 16, 16)
            ixs[b][s] = ixs[b][s] + f * CARD
        pltpu.async_copy(tab_hbm.at[ixs[b]], sbs[b], sgs[b])

    def issue_out(j, b):
        f, b0 = chunk_coords(j)
        pltpu.async_copy(sbs[b], out_hbm.at[N_NUM + f, pl.ds(b0, _CH)],
                         sos[b])

    def wait_gather(b):
        pltpu.make_async_copy(tab_hbm.at[ixs[b]], sbs[b], sgs[b]).wait()

    def wait_out(b):
        pltpu.make_async_copy(sbs[b], out_hbm.at[0, pl.ds(0, _CH)],
                              sos[b]).wait()

    start_gather(0, 0)
    start_gather(1, 1)
    for j in range(_CPT):
        b = j % _NBUF
        wait_gather(b)
        issue_out(j, b)
        nj = j + 2
        if nj < _CPT:
            nb_ = nj % _NBUF
            if nj >= _NBUF:
                wait_out(nb_)
            start_gather(nj, nb_)
    for j in range(_CPT - _NBUF, _CPT):
        wait_out(j % _NBUF)


def _sc_gather(x_cat_t, folded):
    mesh = plsc.VectorSubcoreMesh(core_axis_name="c", subcore_axis_name="s")
    sc = pl.kernel(
        _sc_body,
        mesh=mesh,
        out_type=jax.ShapeDtypeStruct((N_TOK, B, D), jnp.float32),
        scratch_types=(
            [pltpu.VMEM((_CH,), jnp.int32) for _ in range(_NBUF)]
            + [pltpu.VMEM((_CH, D), jnp.float32) for _ in range(_NBUF)]
            + [pltpu.SemaphoreType.DMA for _ in range(2 * _NBUF)]
        ),
    )
    return sc(x_cat_t, folded)


_BB = 1024


def _num_body(out_in_ref, xn_ref, w_ref, nb_ref, out_ref):
    del out_in_ref
    x = xn_ref[0, 0]
    w = w_ref[0, 0]
    nb = nb_ref[0, 0]
    prod = jax.lax.dot_general(
        x[:, None], w[None, :], (((1,), (0,)), ((), ())),
        precision=jax.lax.Precision.HIGHEST,
        preferred_element_type=jnp.float32)
    out_ref[0] = prod + nb[None, :]


def _fill_num(catout, x_num_t3, w3, nb3):
    return pl.pallas_call(
        _num_body,
        grid=(N_NUM, B // _BB),
        in_specs=[
            pl.BlockSpec(memory_space=pl.ANY),
            pl.BlockSpec((1, 1, _BB), lambda t, i: (t, 0, i)),
            pl.BlockSpec((1, 1, D), lambda t, i: (t, 0, 0)),
            pl.BlockSpec((1, 1, D), lambda t, i: (t, 0, 0)),
        ],
        out_specs=pl.BlockSpec((1, _BB, D), lambda t, i: (t, i, 0)),
        out_shape=jax.ShapeDtypeStruct((N_TOK, B, D), jnp.float32),
        input_output_aliases={0: 0},
    )(catout, x_num_t3, w3, nb3)


@jax.jit
def _run(x_num_t3, x_cat_t, w3, nb3, cat_table, cat_bias):
    folded = _fold_table(cat_table, cat_bias)
    catout = _sc_gather(x_cat_t, folded)
    out_tm = _fill_num(catout, x_num_t3, w3, nb3)
    return jnp.transpose(out_tm, (1, 0, 2))


def kernel(x_num, x_cat, num_weight, num_bias, cat_table, cat_bias):
    x_num_t3 = x_num.T[:, None, :]
    x_cat_t = x_cat.astype(jnp.int32).T
    w3 = num_weight[:, None, :]
    nb3 = num_bias[:, None, :]
    return _run(x_num_t3, x_cat_t, w3, nb3, cat_table, cat_bias)


# speedup vs baseline: 1.0582x; 1.0582x over previous
import jax
import jax.numpy as jnp
from jax import lax
from jax.experimental import pallas as pl
from jax.experimental.pallas import tpu as pltpu
from jax.experimental.pallas import tpu_sc as plsc

N_NUM = 13
N_CAT = 26
CARD = 1000
D = 128
B = 4096
N_TOK = N_NUM + N_CAT

try:
    _INFO = plsc.get_sparse_core_info()
    _NC = _INFO.num_cores
    _NS = _INFO.num_subcores
except Exception:
    _NC = 2
    _NS = 16
_NW = _NC * _NS
_CH = 256
_CPT = (N_CAT * B) // _CH // _NW
_NBUF = 3


def _fold_body(tab_ref, bias_ref, out_ref):
    out_ref[:CARD] = tab_ref[:CARD] + bias_ref[0]
    out_ref[CARD:] = tab_ref[CARD:] + bias_ref[1]


def _fold_table(cat_table, cat_bias):
    return pl.pallas_call(
        _fold_body,
        grid=(N_CAT // 2,),
        in_specs=[
            pl.BlockSpec((2 * CARD, D), lambda i: (i, 0)),
            pl.BlockSpec((2, 1, D), lambda i: (i, 0, 0)),
        ],
        out_specs=pl.BlockSpec((2 * CARD, D), lambda i: (i, 0)),
        out_shape=jax.ShapeDtypeStruct((N_CAT * CARD, D), jnp.float32),
    )(cat_table, cat_bias[:, None, :])


def _sc_body(xcatt_hbm, tab_hbm, out_hbm,
             ix0, ix1, ix2, sb0, sb1, sb2,
             sg0, sg1, sg2, so0, so1, so2):
    wid = lax.axis_index("s") * _NC + lax.axis_index("c")
    k0 = wid * _CPT

    ixs = [ix0, ix1, ix2]
    sbs = [sb0, sb1, sb2]
    sgs = [sg0, sg1, sg2]
    sos = [so0, so1, so2]

    def chunk_coords(j):
        k = k0 + j
        f = lax.div(k, jnp.int32(B // _CH))
        b0 = lax.rem(k, jnp.int32(B // _CH)) * _CH
        return f, b0

    def start_gather(j, b):
        f, b0 = chunk_coords(j)
        pltpu.sync_copy(xcatt_hbm.at[f, pl.ds(b0, _CH)], ixs[b])
        for v in range(_CH // 16):
            s = pl.ds(v * 16, 16)
            ixs[b][s] = ixs[b][s] + f * CARD
        pltpu.async_copy(tab_hbm.at[ixs[b]], sbs[b], sgs[b])

    def issue_out(j, b):
        f, b0 = chunk_coords(j)
        pltpu.async_copy(sbs[b], out_hbm.at[N_NUM + f, pl.ds(b0, _CH)],
                         sos[b])

    def wait_gather(b):
        pltpu.make_async_copy(tab_hbm.at[ixs[b]], sbs[b], sgs[b]).wait()

    def wait_out(b):
        pltpu.make_async_copy(sbs[b], out_hbm.at[0, pl.ds(0, _CH)],
                              sos[b]).wait()

    start_gather(0, 0)
    start_gather(1, 1)
    for j in range(_CPT):
        b = j % _NBUF
        wait_gather(b)
        issue_out(j, b)
        nj = j + 2
        if nj < _CPT:
            nb_ = nj % _NBUF
            if nj >= _NBUF:
                wait_out(nb_)
            start_gather(nj, nb_)
    for j in range(_CPT - _NBUF, _CPT):
        wait_out(j % _NBUF)


def _sc_gather(x_cat_t, folded):
    mesh = plsc.VectorSubcoreMesh(core_axis_name="c", subcore_axis_name="s")
    sc = pl.kernel(
        _sc_body,
        mesh=mesh,
        out_type=jax.ShapeDtypeStruct((N_TOK, B, D), jnp.float32),
        scratch_types=(
            [pltpu.VMEM((_CH,), jnp.int32) for _ in range(_NBUF)]
            + [pltpu.VMEM((_CH, D), jnp.float32) for _ in range(_NBUF)]
            + [pltpu.SemaphoreType.DMA for _ in range(2 * _NBUF)]
        ),
    )
    return sc(x_cat_t, folded)


_BB = 1024


def _num_body(out_in_ref, xn_ref, w_ref, nb_ref, out_ref):
    del out_in_ref
    x = xn_ref[0, 0]
    w = w_ref[0, 0]
    nb = nb_ref[0, 0]
    prod = jax.lax.dot_general(
        x[:, None], w[None, :], (((1,), (0,)), ((), ())),
        preferred_element_type=jnp.float32)
    out_ref[0] = prod + nb[None, :]


def _fill_num(catout, x_num_t3, w3, nb3):
    return pl.pallas_call(
        _num_body,
        grid=(N_NUM, B // _BB),
        in_specs=[
            pl.BlockSpec(memory_space=pl.ANY),
            pl.BlockSpec((1, 1, _BB), lambda t, i: (t, 0, i)),
            pl.BlockSpec((1, 1, D), lambda t, i: (t, 0, 0)),
            pl.BlockSpec((1, 1, D), lambda t, i: (t, 0, 0)),
        ],
        out_specs=pl.BlockSpec((1, _BB, D), lambda t, i: (t, i, 0)),
        out_shape=jax.ShapeDtypeStruct((N_TOK, B, D), jnp.float32),
        input_output_aliases={0: 0},
    )(catout, x_num_t3, w3, nb3)


@jax.jit
def _run(x_num_t3, x_cat_t, w3, nb3, cat_table, cat_bias):
    folded = _fold_table(cat_table, cat_bias)
    catout = _sc_gather(x_cat_t, folded)
    out_tm = _fill_num(catout, x_num_t3, w3, nb3)
    return jnp.transpose(out_tm, (1, 0, 2))


def kernel(x_num, x_cat, num_weight, num_bias, cat_table, cat_bias):
    x_num_t3 = x_num.T[:, None, :]
    x_cat_t = x_cat.astype(jnp.int32).T
    w3 = num_weight[:, None, :]
    nb3 = num_bias[:, None, :]
    return _run(x_num_t3, x_cat_t, w3, nb3, cat_table, cat_bias)


# speedup vs baseline: 1.1357x; 1.0732x over previous
import jax
import jax.numpy as jnp
from jax import lax
from jax.experimental import pallas as pl
from jax.experimental.pallas import tpu as pltpu
from jax.experimental.pallas import tpu_sc as plsc

N_NUM = 13
N_CAT = 26
CARD = 1000
D = 128
B = 4096
N_TOK = N_NUM + N_CAT

try:
    _INFO = plsc.get_sparse_core_info()
    _NC = _INFO.num_cores
    _NS = _INFO.num_subcores
except Exception:
    _NC = 2
    _NS = 16
_NW = _NC * _NS
_CH = 256
_NCHUNK = N_TOK * (B // _CH)
_JMAX = 20
_NBUF = 3


def _fold_body(tab_ref, bias_ref, out_ref):
    out_ref[:CARD] = tab_ref[:CARD] + bias_ref[0]
    out_ref[CARD:] = tab_ref[CARD:] + bias_ref[1]


def _fold_table(cat_table, cat_bias):
    return pl.pallas_call(
        _fold_body,
        grid=(N_CAT // 2,),
        in_specs=[
            pl.BlockSpec((2 * CARD, D), lambda i: (i, 0)),
            pl.BlockSpec((2, 1, D), lambda i: (i, 0, 0)),
        ],
        out_specs=pl.BlockSpec((2 * CARD, D), lambda i: (i, 0)),
        out_shape=jax.ShapeDtypeStruct((N_CAT * CARD, D), jnp.float32),
    )(cat_table, cat_bias[:, None, :])


def _sc_body(xnumt_hbm, xcatt_hbm, w_hbm, nb_hbm, tab_hbm, out_hbm,
             w_v, nb_v, xb_v, ix0, ix1, ix2, sb0, sb1, sb2,
             sg0, sg1, sg2, so0, so1, so2):
    wid = lax.axis_index("s") * _NC + lax.axis_index("c")
    p = lax.div(wid, jnp.int32(16))
    b0 = lax.rem(wid, jnp.int32(16)) * _CH

    ixs = [ix0, ix1, ix2]
    sbs = [sb0, sb1, sb2]
    sgs = [sg0, sg1, sg2]
    sos = [so0, so1, so2]

    pltpu.sync_copy(w_hbm, w_v)
    pltpu.sync_copy(nb_hbm, nb_v)

    def t_of(j):
        return 2 * j + p

    def wait_out(b):
        pltpu.make_async_copy(sbs[b], out_hbm.at[0, pl.ds(0, _CH)],
                              sos[b]).wait()

    def wait_gather(b):
        pltpu.make_async_copy(tab_hbm.at[ixs[b]], sbs[b], sgs[b]).wait()

    def issue_out(j, b):
        pltpu.async_copy(sbs[b], out_hbm.at[t_of(j), pl.ds(b0, _CH)],
                         sos[b])

    def prep_gather(j, b):
        f = t_of(j) - N_NUM
        pltpu.sync_copy(xcatt_hbm.at[f, pl.ds(b0, _CH)], ixs[b])
        for v in range(_CH // 16):
            s = pl.ds(v * 16, 16)
            ixs[b][s] = ixs[b][s] + f * CARD
        pltpu.async_copy(tab_hbm.at[ixs[b]], sbs[b], sgs[b])

    def do_num(j, b):
        t = t_of(j)
        pltpu.sync_copy(xnumt_hbm.at[t, pl.ds(b0, _CH)], xb_v)
        wrows = [w_v[t, pl.ds(i * 16, 16)] for i in range(D // 16)]
        nrows = [nb_v[t, pl.ds(i * 16, 16)] for i in range(D // 16)]

        def grp(g, carry, b=b):
            xv = xb_v[pl.ds(g * 16, 16)]
            for l in range(16):
                xs = xv[l]
                r = g * 16 + l
                for i in range(D // 16):
                    sbs[b][r, pl.ds(i * 16, 16)] = (
                        xs * wrows[i] + nrows[i])
            return carry

        lax.fori_loop(0, _CH // 16, grp, 0)
        issue_out(j, b)

    def do_cat(j, b):
        wait_gather(b)
        issue_out(j, b)

    for j in range(_JMAX):
        b = j % _NBUF
        if j <= 5:
            if j >= _NBUF:
                wait_out(b)
            do_num(j, b)
        elif j == 6:
            @pl.when(p == 0)
            def _(b=b):
                wait_out(b)
                do_num(6, b)

            @pl.when(p == 1)
            def _(b=b):
                do_cat(6, b)
        elif j == _JMAX - 1:
            @pl.when(p == 0)
            def _(b=b):
                do_cat(_JMAX - 1, b)
        else:
            do_cat(j, b)

        jp = j + 2
        bp = jp % _NBUF
        if jp == 6:
            @pl.when(p == 1)
            def _(bp=bp):
                wait_out(bp)
                prep_gather(6, bp)
        elif 7 <= jp <= _JMAX - 2:
            wait_out(bp)
            prep_gather(jp, bp)
        elif jp == _JMAX - 1:
            @pl.when(p == 0)
            def _(bp=bp):
                wait_out(bp)
                prep_gather(_JMAX - 1, bp)

    wait_out(2)
    wait_out(0)
    wait_out(1)


@jax.jit
def _run(x_num_t, x_cat_t, w_pad, nb_pad, cat_table, cat_bias):
    folded = _fold_table(cat_table, cat_bias)
    mesh = plsc.VectorSubcoreMesh(core_axis_name="c", subcore_axis_name="s")
    sc = pl.kernel(
        _sc_body,
        mesh=mesh,
        out_type=jax.ShapeDtypeStruct((N_TOK, B, D), jnp.float32),
        scratch_types=(
            [pltpu.VMEM((16, D), jnp.float32),
             pltpu.VMEM((16, D), jnp.float32),
             pltpu.VMEM((_CH,), jnp.float32)]
            + [pltpu.VMEM((_CH,), jnp.int32) for _ in range(_NBUF)]
            + [pltpu.VMEM((_CH, D), jnp.float32) for _ in range(_NBUF)]
            + [pltpu.SemaphoreType.DMA for _ in range(2 * _NBUF)]
        ),
    )
    out_tm = sc(x_num_t, x_cat_t, w_pad, nb_pad, folded)
    return jnp.transpose(out_tm, (1, 0, 2))


def kernel(x_num, x_cat, num_weight, num_bias, cat_table, cat_bias):
    x_num_t = jnp.pad(x_num.T, ((0, 16 - N_NUM), (0, 0)))
    x_cat_t = x_cat.astype(jnp.int32).T
    w_pad = jnp.pad(num_weight, ((0, 16 - N_NUM), (0, 0)))
    nb_pad = jnp.pad(num_bias, ((0, 16 - N_NUM), (0, 0)))
    return _run(x_num_t, x_cat_t, w_pad, nb_pad, cat_table, cat_bias)
